# Initial kernel scaffold; baseline (speedup 1.0000x reference)
import jax
import jax.numpy as jnp
from jax.experimental import pallas as pl


def kernel(atom_xyz, atom_types, res_xyz, surf_xyz, surf_curvs, params):
    raise NotImplementedError("write your pallas kernel here")


# speedup vs baseline: 6.4176x; 6.4176x over previous
import functools

import jax
import jax.numpy as jnp
from jax import lax
from jax.experimental import pallas as pl
from jax.experimental.pallas import tpu as pltpu
from jax.experimental.pallas import tpu_sc as plsc

F32 = jnp.float32


def _ru(x, m):
    return (x + m - 1) // m * m


def _pad_rows(x, n, val=0.0):
    if x.shape[0] == n:
        return x
    pad = jnp.full((n - x.shape[0],) + x.shape[1:], val, x.dtype)
    return jnp.concatenate([x, pad], 0)


def _pad_cols(x, d, val=0.0):
    if x.shape[-1] == d:
        return x
    pad = jnp.full(x.shape[:-1] + (d - x.shape[-1],), val, x.dtype)
    return jnp.concatenate([x, pad], -1)


def _knn(q, s, k, want_d):
    Nq, Ns = q.shape[0], s.shape[0]
    TQ = 256
    Nqp, NSp = _ru(Nq, TQ), _ru(Ns, 128)
    qp = _pad_rows(q, Nqp)
    st = _pad_rows(s, NSp, 1e6).T

    out_shape = [jax.ShapeDtypeStruct((Nqp, k), jnp.int32)]
    out_specs = [pl.BlockSpec((TQ, k), lambda t: (t, 0))]
    if want_d:
        out_shape.append(jax.ShapeDtypeStruct((Nqp, k), F32))
        out_specs.append(pl.BlockSpec((TQ, k), lambda t: (t, 0)))

    def body(q_ref, st_ref, idx_ref, *maybe_d):
        qb = q_ref[...]
        stv = st_ref[...]
        s2 = jnp.sum(stv * stv, axis=0, keepdims=True)
        q2 = jnp.sum(qb * qb, axis=1, keepdims=True)
        d = q2 + s2 - 2.0 * jax.lax.dot(qb, stv,
                                        precision=jax.lax.Precision.DEFAULT,
                                        preferred_element_type=F32)
        iota = jax.lax.broadcasted_iota(jnp.int32, (TQ, NSp), 1)
        icols, vcols = [], []
        for j in range(k):
            v = jnp.min(d, axis=1, keepdims=True)
            i = jnp.min(jnp.where(d == v, iota, jnp.int32(2**30)), axis=1,
                        keepdims=True)
            icols.append(i)
            vcols.append(v)
            if j < k - 1:
                d = jnp.where(iota == i, F32(3.0e38), d)
        idx_ref[...] = jnp.concatenate(icols, axis=1)
        if maybe_d:
            maybe_d[0][...] = jnp.maximum(jnp.concatenate(vcols, 1), 0.0)

    res = pl.pallas_call(
        body,
        grid=(Nqp // TQ,),
        in_specs=[pl.BlockSpec((TQ, 3), lambda t: (t, 0)),
                  pl.BlockSpec((3, NSp), lambda t: (0, 0))],
        out_specs=out_specs,
        out_shape=out_shape,
        interpret=False,
    )(qp, st)
    if want_d:
        return res[1][:Nq], res[0][:Nq]
    return res[0][:Nq]


@functools.lru_cache(maxsize=None)
def _gather_call(V, D, B, CB, nch):
    NW = 32
    mesh = plsc.VectorSubcoreMesh(core_axis_name="c", subcore_axis_name="s")

    @functools.partial(
        pl.kernel, mesh=mesh,
        out_type=jax.ShapeDtypeStruct((B, D), F32),
        scratch_types=[pltpu.VMEM((CB,), jnp.int32),
                       pltpu.VMEM((CB, D), F32),
                       pltpu.SemaphoreType.DMA],
        compiler_params=pltpu.CompilerParams(use_tc_tiling_on_sc=False),
    )
    def k(table_hbm, idx_hbm, out_hbm, idx_v, rows_v, sem):
        wid = lax.axis_index("s") * 2 + lax.axis_index("c")
        base = wid * (nch * CB)
        for c in range(nch):
            off = base + c * CB
            pltpu.sync_copy(idx_hbm.at[pl.ds(off, CB)], idx_v)
            pltpu.async_copy(table_hbm.at[idx_v], rows_v, sem).wait()
            pltpu.sync_copy(rows_v, out_hbm.at[pl.ds(off, CB)])

    return k


def _gather_rows(table, idx):
    B0 = idx.shape[0]
    B = _ru(B0, 256)
    idxp = _pad_rows(idx, B)
    V, D = table.shape
    NW = 32
    bpw = B // NW
    cap = min(1000, (400_000 // (D * 4)) // 8 * 8)
    CB = 8
    for c in range(8, min(bpw, cap) + 1, 8):
        if bpw % c == 0:
            CB = c
    nch = bpw // CB
    out = _gather_call(V, D, B, CB, nch)(table, idxp)
    return out[:B0]


def _sa_max(g, qxyz, W1, b1, W2, b2, fq=None, Wsc=None, bsc=None):
    K_, Np, D = g.shape
    H = W2.shape[1]
    TN = min(512, Np)
    b1r, b2r = b1.reshape(1, -1), b2.reshape(1, -1)
    ins = [g, qxyz, W1, b1r, W2, b2r]
    in_specs = [
        pl.BlockSpec((K_, TN, D), lambda t: (0, t, 0)),
        pl.BlockSpec((TN, 3), lambda t: (t, 0)),
        pl.BlockSpec(W1.shape, lambda t: (0, 0)),
        pl.BlockSpec(b1r.shape, lambda t: (0, 0)),
        pl.BlockSpec(W2.shape, lambda t: (0, 0)),
        pl.BlockSpec(b2r.shape, lambda t: (0, 0)),
    ]
    has_sc = fq is not None
    if has_sc:
        bscr = bsc.reshape(1, -1)
        ins += [fq, Wsc, bscr]
        in_specs += [
            pl.BlockSpec((TN, fq.shape[1]), lambda t: (t, 0)),
            pl.BlockSpec(Wsc.shape, lambda t: (0, 0)),
            pl.BlockSpec(bscr.shape, lambda t: (0, 0)),
        ]

    def body(g_ref, q_ref, W1_ref, b1_ref, W2_ref, b2_ref, *rest):
        if has_sc:
            fq_ref, Wsc_ref, bsc_ref, o_ref = rest
        else:
            (o_ref,) = rest
        q = q_ref[...]
        qpad = jnp.concatenate([q, jnp.zeros((TN, D - 3), F32)], axis=1)
        W1v, b1v = W1_ref[...], b1_ref[...]
        W2v, b2v = W2_ref[...], b2_ref[...]
        acc = None
        for j in range(K_):
            x = g_ref[j] - qpad
            z = jnp.maximum(jnp.dot(x, W1v, preferred_element_type=F32) + b1v, 0.0)
            y = jnp.dot(z, W2v, preferred_element_type=F32) + b2v
            acc = y if acc is None else jnp.maximum(acc, y)
        acc = jnp.maximum(acc, 0.0)
        if has_sc:
            sc = jnp.dot(fq_ref[...], Wsc_ref[...], preferred_element_type=F32)
            acc = jnp.maximum(acc + sc + bsc_ref[...], 0.0)
        o_ref[...] = acc

    return pl.pallas_call(
        body,
        grid=(Np // TN,),
        in_specs=in_specs,
        out_specs=pl.BlockSpec((TN, H), lambda t: (t, 0)),
        out_shape=jax.ShapeDtypeStruct((Np, H), F32),
        interpret=False,
    )(*ins)


def _interp_mlp(g3, dmat, f1, layers):
    _, Np, D2 = g3.shape
    TN = min(512, Np)
    ins = [g3, dmat]
    in_specs = [pl.BlockSpec((3, TN, D2), lambda t: (0, t, 0)),
                pl.BlockSpec((TN, 3), lambda t: (t, 0))]
    if f1 is not None:
        ins.append(f1)
        in_specs.append(pl.BlockSpec((TN, f1.shape[1]), lambda t: (t, 0)))
    wb = []
    for (W, b) in layers:
        br = b.reshape(1, -1)
        ins += [W, br]
        in_specs += [pl.BlockSpec(W.shape, lambda t: (0, 0)),
                     pl.BlockSpec(br.shape, lambda t: (0, 0))]
        wb.append((W.shape, br.shape))
    Hout = layers[-1][0].shape[1] if layers else D2

    def body(g_ref, d_ref, *rest):
        o_ref = rest[-1]
        pos = 0
        if f1 is not None:
            f1_ref = rest[pos]
            pos += 1
        d = d_ref[...]
        w = 1.0 / (d + 1e-8)
        wn = w / jnp.sum(w, axis=1, keepdims=True)
        ae = (wn[:, 0:1] * g_ref[0] + wn[:, 1:2] * g_ref[1]
              + wn[:, 2:3] * g_ref[2])
        x = jnp.concatenate([f1_ref[...], ae], axis=1) if f1 is not None else ae
        for _ in layers:
            Wv, bv = rest[pos][...], rest[pos + 1][...]
            pos += 2
            x = jnp.maximum(jnp.dot(x, Wv, preferred_element_type=F32) + bv, 0.0)
        o_ref[...] = x

    return pl.pallas_call(
        body,
        grid=(Np // TN,),
        in_specs=in_specs,
        out_specs=pl.BlockSpec((TN, Hout), lambda t: (t, 0)),
        out_shape=jax.ShapeDtypeStruct((Np, Hout), F32),
        interpret=False,
    )(*ins)


def _mlp_pallas(x, layers):
    Np = x.shape[0]
    TN = min(512, Np)
    ins = [x]
    in_specs = [pl.BlockSpec((TN, x.shape[1]), lambda t: (t, 0))]
    for (W, b) in layers:
        br = b.reshape(1, -1)
        ins += [W, br]
        in_specs += [pl.BlockSpec(W.shape, lambda t: (0, 0)),
                     pl.BlockSpec(br.shape, lambda t: (0, 0))]
    Hout = layers[-1][0].shape[1]

    def body(x_ref, *rest):
        o_ref = rest[-1]
        xv = x_ref[...]
        pos = 0
        for _ in layers:
            Wv, bv = rest[pos][...], rest[pos + 1][...]
            pos += 2
            xv = jnp.maximum(jnp.dot(xv, Wv, preferred_element_type=F32) + bv, 0.0)
        o_ref[...] = xv

    return pl.pallas_call(
        body,
        grid=(Np // TN,),
        in_specs=in_specs,
        out_specs=pl.BlockSpec((TN, Hout), lambda t: (t, 0)),
        out_shape=jax.ShapeDtypeStruct((Np, Hout), F32),
        interpret=False,
    )(*ins)


def _layers(m):
    return list(zip(m['Ws'], m['bs']))


def _flat_kmajor(idx):
    return idx.T.reshape(-1)


def _sa_stage(feats_tab, xyz_src, qxyz, idx, m, D, fq=None, Wsc=None, bsc=None):
    N, K_ = idx.shape
    Np = _ru(N, 512) if N >= 512 else 512
    table = _pad_cols(jnp.concatenate([xyz_src, feats_tab], axis=1), D)
    rows = _gather_rows(table, _flat_kmajor(idx))
    g = rows.reshape(K_, N, D)
    if Np != N:
        g = jnp.concatenate([g, jnp.zeros((K_, Np - N, D), F32)], axis=1)
    qp = _pad_rows(qxyz, Np)
    Ws, bs = m['Ws'], m['bs']
    W1 = _pad_rows(Ws[0], D)
    if fq is not None:
        Dq = _ru(fq.shape[1], 8)
        fqp = _pad_rows(_pad_cols(fq, Dq), Np)
        Wscp = _pad_rows(Wsc, Dq)
        out = _sa_max(g, qp, W1, bs[0], Ws[1], bs[1], fqp, Wscp, bsc)
    else:
        out = _sa_max(g, qp, W1, bs[0], Ws[1], bs[1])
    return out[:N]


def _interp_stage(table, idx, d, f1=None, layers=()):
    N = idx.shape[0]
    Np = _ru(N, 512) if N >= 512 else 512
    rows = _gather_rows(table, _flat_kmajor(idx))
    D2 = table.shape[1]
    g3 = rows.reshape(3, N, D2)
    if Np != N:
        g3 = jnp.concatenate([g3, jnp.zeros((3, Np - N, D2), F32)], axis=1)
    dp = _pad_rows(d, Np)
    f1p = _pad_rows(f1, Np) if f1 is not None else None
    out = _interp_mlp(g3, dp, f1p, list(layers))
    return out[:N]


def kernel(atom_xyz, atom_types, res_xyz, surf_xyz, surf_curvs, params):
    p = params
    NA = atom_xyz.shape[0]
    NS = surf_xyz.shape[0]

    f0 = _mlp_pallas(_pad_rows(atom_types, _ru(NA, 512)),
                     _layers(p['atom_linear']))[:NA]

    idxAA = _knn(atom_xyz, atom_xyz, 16, False)
    f1 = _sa_stage(f0, atom_xyz, atom_xyz, idxAA, p['atom_sa'], 48)

    idxRA = _knn(res_xyz, atom_xyz, 16, False)
    f2 = _sa_stage(f1, atom_xyz, res_xyz, idxRA, p['atom_sa_ds'], 48)

    dAR, idxAR = _knn(atom_xyz, res_xyz, 3, True)
    f01 = jnp.concatenate([f0, f1], axis=1)
    f3 = _interp_stage(f2, idxAR, dAR, f01, _layers(p['atom_fp']))

    dSA, idxSA = _knn(surf_xyz, atom_xyz, 3, True)
    ae = _interp_stage(f1, idxSA, dSA)
    sf = jnp.concatenate([surf_curvs, ae], axis=1)

    idxSS = _knn(surf_xyz, surf_xyz, 16, False)
    hm = p['head_sa']
    sf1 = _sa_stage(sf, surf_xyz, surf_xyz, idxSS, hm['mlp'], 48,
                    fq=sf, Wsc=hm['sc']['W'], bsc=hm['sc']['b'])

    sx1 = surf_xyz[::4]
    sfp = sf1[::4]
    d1, i1 = _knn(sx1, res_xyz, 3, True)
    ae1 = _interp_stage(f2, i1, d1)
    c0 = jnp.concatenate([sfp, ae1], axis=1)
    idx11 = _knn(sx1, sx1, 16, False)
    m0 = p['sa0']
    sf2 = _sa_stage(c0, sx1, sx1, idx11, m0['mlp'], 80,
                    fq=c0, Wsc=m0['sc']['W'], bsc=m0['sc']['b'])

    sx2 = sx1[::4]
    sfp2 = sf2[::4]
    d2, i2 = _knn(sx2, res_xyz, 3, True)
    ae2 = _interp_stage(f2, i2, d2)
    c1 = jnp.concatenate([sfp2, ae2], axis=1)
    idx22 = _knn(sx2, sx2, 16, False)
    m1 = p['sa1']
    sf3 = _sa_stage(c1, sx2, sx2, idx22, m1['mlp'], 112,
                    fq=c1, Wsc=m1['sc']['W'], bsc=m1['sc']['b'])

    d3, i3 = _knn(sx1, sx2, 3, True)
    sfA = _interp_stage(sf3, i3, d3, sf2, _layers(p['fp0']))

    d4, i4 = _knn(surf_xyz, sx1, 3, True)
    sfB = _interp_stage(sfA, i4, d4, sf1, _layers(p['fp1']))

    ae3 = _interp_stage(f3, idxSA, dSA)

    out = jnp.concatenate([sfB, ae3], axis=1)
    return jnp.transpose(out)[None, :, :]
